# baseline (speedup 1.0000x reference)
import functools

import jax
import jax.numpy as jnp
from jax import lax
from jax.experimental import pallas as pl
from jax.experimental.pallas import tpu as pltpu
from jax.experimental.pallas import tpu_sc as plsc

_NC = 2
_NS = 16
_NW = _NC * _NS
_B, _F, _H, _W = 4, 16, 512, 512
_PLANES_PER_W = (_B * _F) // _NW
_ROWS = 64
_CHUNKS_PER_PLANE = _H // _ROWS
_NCHUNK = _PLANES_PER_W * _CHUNKS_PER_PLANE


def _sc_body(x_hbm, out_hbm, buf0, buf1, acc_ref, sem0, sem1):
    cid = lax.axis_index("c")
    sid = lax.axis_index("s")
    wid = sid * _NC + cid

    bufs = (buf0, buf1)
    sems = (sem0, sem1)

    def chunk_src(k):
        plane = wid * _PLANES_PER_W + (k // _CHUNKS_PER_PLANE)
        b = plane // _F
        f = plane % _F
        r0 = (k % _CHUNKS_PER_PLANE) * _ROWS
        return x_hbm.at[b, f, pl.ds(r0, _ROWS), :]

    pltpu.async_copy(chunk_src(0), buf0, sem0)

    accs = (jnp.zeros((16,), jnp.float32),) * 4
    for k in range(_NCHUNK):
        cur = bufs[k % 2]
        if k + 1 < _NCHUNK:
            pltpu.async_copy(chunk_src(k + 1), bufs[(k + 1) % 2], sems[(k + 1) % 2])
        pltpu.make_async_copy(chunk_src(k), cur, sems[k % 2]).wait()

        def row_sum(r, a, cur=cur):
            for j in range(_W // 16):
                a = a[:j % 4] + (a[j % 4] + cur[r, pl.ds(j * 16, 16)],) + a[j % 4 + 1:]
            return a

        accs = lax.fori_loop(0, _ROWS, row_sum, accs)

    acc_ref[...] = (accs[0] + accs[1]) + (accs[2] + accs[3])
    pltpu.sync_copy(acc_ref, out_hbm.at[wid])


@jax.jit
def _sc_sum(x):
    run = functools.partial(
        pl.kernel,
        mesh=plsc.VectorSubcoreMesh(core_axis_name="c", subcore_axis_name="s"),
        out_type=jax.ShapeDtypeStruct((_NW, 16), jnp.float32),
        scratch_types=[
            pltpu.VMEM((_ROWS, _W), jnp.float32),
            pltpu.VMEM((_ROWS, _W), jnp.float32),
            pltpu.VMEM((16,), jnp.float32),
            pltpu.SemaphoreType.DMA,
            pltpu.SemaphoreType.DMA,
        ],
    )(_sc_body)
    return run(x)


def kernel(prediction, target):
    del target
    partials = _sc_sum(prediction)
    return jnp.sum(partials)


# speedup vs baseline: 1.2791x; 1.2791x over previous
import functools

import jax
import jax.numpy as jnp
from jax import lax
from jax.experimental import pallas as pl
from jax.experimental.pallas import tpu as pltpu
from jax.experimental.pallas import tpu_sc as plsc

_B, _F, _H, _W = 4, 16, 512, 512


_NC = 2
_NS = 16
_NW = _NC * _NS
_F_TC = 12
_F_SC = _F - _F_TC
_PLANES = _B * _F_SC
_ROWS_PER_W = (_PLANES * _H) // _NW
_ROWS = 64
_NCHUNK = _ROWS_PER_W // _ROWS


def _sc_body(x_hbm, out_hbm, buf0, buf1, acc_ref, sem0, sem1):
    cid = lax.axis_index("c")
    sid = lax.axis_index("s")
    wid = sid * _NC + cid

    bufs = (buf0, buf1)
    sems = (sem0, sem1)

    plane = wid // 2
    b = plane // _F_SC
    f = _F_TC + plane % _F_SC
    row_base = (wid % 2) * _ROWS_PER_W

    def chunk_src(k):
        return x_hbm.at[b, f, pl.ds(row_base + k * _ROWS, _ROWS), :]

    pltpu.async_copy(chunk_src(0), buf0, sem0)

    accs = (jnp.zeros((16,), jnp.float32),) * 4
    for k in range(_NCHUNK):
        cur = bufs[k % 2]
        if k + 1 < _NCHUNK:
            pltpu.async_copy(chunk_src(k + 1), bufs[(k + 1) % 2], sems[(k + 1) % 2])
        pltpu.make_async_copy(chunk_src(k), cur, sems[k % 2]).wait()

        def row_sum(r, a, cur=cur):
            for j in range(_W // 16):
                a = a[:j % 4] + (a[j % 4] + cur[r, pl.ds(j * 16, 16)],) + a[j % 4 + 1:]
            return a

        accs = lax.fori_loop(0, _ROWS, row_sum, accs)

    acc_ref[...] = (accs[0] + accs[1]) + (accs[2] + accs[3])
    pltpu.sync_copy(acc_ref, out_hbm.at[wid])


_sc_sum = functools.partial(
    pl.kernel,
    mesh=plsc.VectorSubcoreMesh(core_axis_name="c", subcore_axis_name="s"),
    out_type=jax.ShapeDtypeStruct((_NW, 16), jnp.float32),
    scratch_types=[
        pltpu.VMEM((_ROWS, _W), jnp.float32),
        pltpu.VMEM((_ROWS, _W), jnp.float32),
        pltpu.VMEM((16,), jnp.float32),
        pltpu.SemaphoreType.DMA,
        pltpu.SemaphoreType.DMA,
    ],
)(_sc_body)


_FB = 6


def _tc_body(x_ref, o_ref, acc_ref):
    i = pl.program_id(0)
    j = pl.program_id(1)

    @pl.when((i == 0) & (j == 0))
    def _init():
        acc_ref[...] = jnp.zeros_like(acc_ref)

    x = x_ref[...].reshape(-1, 8, _W)
    acc_ref[...] += jnp.sum(x, axis=0)

    @pl.when((i == pl.num_programs(0) - 1) & (j == pl.num_programs(1) - 1))
    def _fini():
        o_ref[0, 0] = jnp.sum(acc_ref[...])


def _tc_sum(x):
    return pl.pallas_call(
        _tc_body,
        grid=(_B, _F_TC // _FB),
        in_specs=[pl.BlockSpec((1, _FB, _H, _W), lambda i, j: (i, j, 0, 0))],
        out_specs=pl.BlockSpec(memory_space=pltpu.SMEM),
        out_shape=jax.ShapeDtypeStruct((1, 1), jnp.float32),
        scratch_shapes=[pltpu.VMEM((8, _W), jnp.float32)],
    )(x)


def kernel(prediction, target):
    del target
    sc_partials = _sc_sum(prediction)
    tc_partial = _tc_sum(prediction)
    return tc_partial[0, 0] + jnp.sum(sc_partials)


# speedup vs baseline: 1.2851x; 1.0047x over previous
import functools

import jax
import jax.numpy as jnp
from jax import lax
from jax.experimental import pallas as pl
from jax.experimental.pallas import tpu as pltpu
from jax.experimental.pallas import tpu_sc as plsc

_B, _F, _H, _W = 4, 16, 512, 512


_NC = 2
_NS = 16
_NW = _NC * _NS
_F_TC = 14
_F_SC = _F - _F_TC
_PLANES = _B * _F_SC
_ROWS_PER_W = (_PLANES * _H) // _NW
_ROWS = 64
_NCHUNK = _ROWS_PER_W // _ROWS


def _sc_body(x_hbm, out_hbm, buf0, buf1, acc_ref, sem0, sem1):
    cid = lax.axis_index("c")
    sid = lax.axis_index("s")
    wid = sid * _NC + cid

    bufs = (buf0, buf1)
    sems = (sem0, sem1)

    wpp = _NW // _PLANES
    plane = wid // wpp
    b = plane // _F_SC
    f = _F_TC + plane % _F_SC
    row_base = (wid % wpp) * _ROWS_PER_W

    def chunk_src(k):
        return x_hbm.at[b, f, pl.ds(row_base + k * _ROWS, _ROWS), :]

    pltpu.async_copy(chunk_src(0), buf0, sem0)

    accs = (jnp.zeros((16,), jnp.float32),) * 4
    for k in range(_NCHUNK):
        cur = bufs[k % 2]
        if k + 1 < _NCHUNK:
            pltpu.async_copy(chunk_src(k + 1), bufs[(k + 1) % 2], sems[(k + 1) % 2])
        pltpu.make_async_copy(chunk_src(k), cur, sems[k % 2]).wait()

        def row_sum(r, a, cur=cur):
            for j in range(_W // 16):
                a = a[:j % 4] + (a[j % 4] + cur[r, pl.ds(j * 16, 16)],) + a[j % 4 + 1:]
            return a

        accs = lax.fori_loop(0, _ROWS, row_sum, accs)

    acc_ref[...] = (accs[0] + accs[1]) + (accs[2] + accs[3])
    pltpu.sync_copy(acc_ref, out_hbm.at[wid])


_sc_sum = functools.partial(
    pl.kernel,
    mesh=plsc.VectorSubcoreMesh(core_axis_name="c", subcore_axis_name="s"),
    out_type=jax.ShapeDtypeStruct((_NW, 16), jnp.float32),
    scratch_types=[
        pltpu.VMEM((_ROWS, _W), jnp.float32),
        pltpu.VMEM((_ROWS, _W), jnp.float32),
        pltpu.VMEM((16,), jnp.float32),
        pltpu.SemaphoreType.DMA,
        pltpu.SemaphoreType.DMA,
    ],
)(_sc_body)


_FB = 7


def _tc_body(x_ref, o_ref, acc_ref):
    i = pl.program_id(0)
    j = pl.program_id(1)

    @pl.when((i == 0) & (j == 0))
    def _init():
        acc_ref[...] = jnp.zeros_like(acc_ref)

    x = x_ref[...].reshape(-1, 8, _W)
    acc_ref[...] += jnp.sum(x, axis=0)

    @pl.when((i == pl.num_programs(0) - 1) & (j == pl.num_programs(1) - 1))
    def _fini():
        o_ref[0, 0] = jnp.sum(acc_ref[...])


def _tc_sum(x):
    return pl.pallas_call(
        _tc_body,
        grid=(_B, _F_TC // _FB),
        in_specs=[pl.BlockSpec((1, _FB, _H, _W), lambda i, j: (i, j, 0, 0))],
        out_specs=pl.BlockSpec(memory_space=pltpu.SMEM),
        out_shape=jax.ShapeDtypeStruct((1, 1), jnp.float32),
        scratch_shapes=[pltpu.VMEM((8, _W), jnp.float32)],
    )(x)


def kernel(prediction, target):
    del target
    sc_partials = _sc_sum(prediction)
    tc_partial = _tc_sum(prediction)
    return tc_partial[0, 0] + jnp.sum(sc_partials)


# speedup vs baseline: 2.3712x; 1.8452x over previous
import jax
import jax.numpy as jnp
from jax.experimental import pallas as pl
from jax.experimental.pallas import tpu as pltpu

_FB = 8


def _sum_body(a_ref, b_ref, o_ref, acc_ref):
    i = pl.program_id(0)

    @pl.when(i == 0)
    def _init():
        acc_ref[...] = jnp.zeros_like(acc_ref)

    a = a_ref[...].reshape(-1, 8, 512)
    b = b_ref[...].reshape(-1, 8, 512)
    acc_ref[...] += jnp.sum(a, axis=0) + jnp.sum(b, axis=0)

    @pl.when(i == pl.num_programs(0) - 1)
    def _fini():
        o_ref[0, 0] = jnp.sum(acc_ref[...])


def kernel(prediction, target):
    del target
    B, F, H, W = prediction.shape
    out = pl.pallas_call(
        _sum_body,
        grid=(B,),
        in_specs=[
            pl.BlockSpec((1, _FB, H, W), lambda i: (i, 0, 0, 0)),
            pl.BlockSpec((1, _FB, H, W), lambda i: (i, 1, 0, 0)),
        ],
        out_specs=pl.BlockSpec(memory_space=pltpu.SMEM),
        out_shape=jax.ShapeDtypeStruct((1, 1), jnp.float32),
        scratch_shapes=[pltpu.VMEM((8, 512), jnp.float32)],
    )(prediction, prediction)
    return out[0, 0]


# speedup vs baseline: 2.4096x; 1.0162x over previous
import jax
import jax.numpy as jnp
from jax.experimental import pallas as pl
from jax.experimental.pallas import tpu as pltpu

_FB = 8


def _sum_body(x_ref, o_ref, acc_ref):
    i = pl.program_id(0)
    j = pl.program_id(1)

    @pl.when((i == 0) & (j == 0))
    def _init():
        acc_ref[...] = jnp.zeros_like(acc_ref)

    x = x_ref[...].reshape(-1, 8, 512)
    acc_ref[...] += jnp.sum(x, axis=0)

    @pl.when((i == pl.num_programs(0) - 1) & (j == pl.num_programs(1) - 1))
    def _fini():
        o_ref[0, 0] = jnp.sum(acc_ref[...])


def kernel(prediction, target):
    del target
    B, F, H, W = prediction.shape
    out = pl.pallas_call(
        _sum_body,
        grid=(B, F // _FB),
        in_specs=[pl.BlockSpec((1, _FB, H, W), lambda i, j: (i, j, 0, 0))],
        out_specs=pl.BlockSpec(memory_space=pltpu.SMEM),
        out_shape=jax.ShapeDtypeStruct((1, 1), jnp.float32),
        scratch_shapes=[pltpu.VMEM((8, W), jnp.float32)],
    )(prediction)
    return out[0, 0]
